# Initial kernel scaffold; baseline (speedup 1.0000x reference)
import jax
import jax.numpy as jnp
from jax.experimental import pallas as pl


def kernel(P1, P2, X1, S2, W, b):
    raise NotImplementedError("write your pallas kernel here")


# speedup vs baseline: 2.5355x; 2.5355x over previous
import functools

import jax
import jax.numpy as jnp
from jax.experimental import pallas as pl
from jax.experimental.pallas import tpu as pltpu

RADIUS = 0.2
NS = 32
C = 64
QB = 128
CB = 128
NEG = -3.0e38


def _pre_body(s2t_ref, x1t_ref, p1p_ref, p2p_ref, wst_ref, wxt_ref,
              wdt_ref, b_ref, g_ref, base_ref):
    s2t = s2t_ref[0]
    x1t = x1t_ref[0]
    p1p = p1p_ref[0]
    p2p = p2p_ref[0]
    wst = wst_ref[...]
    wxt = wxt_ref[...]
    wdt = wdt_ref[...]
    bb = b_ref[...]
    f32 = jnp.float32
    g = (jnp.dot(s2t, wst, preferred_element_type=f32)
         + jnp.dot(p2p, wdt, preferred_element_type=f32))
    base = (jnp.dot(x1t, wxt, preferred_element_type=f32)
            - jnp.dot(p1p, wdt, preferred_element_type=f32) + bb)
    g_ref[0] = g
    base_ref[0] = base


def _bq_body(p1p_ref, p2t_ref, g_ref, basen_ref, out_ref):
    p1 = p1p_ref[0]
    n = p2t_ref.shape[2]
    rsq = jnp.float32(RADIUS * RADIUS)

    ri = jax.lax.broadcasted_iota(jnp.int32, (CB, CB), 0)
    ci = jax.lax.broadcasted_iota(jnp.int32, (CB, CB), 1)
    tri = (ri <= ci).astype(jnp.float32)

    nchunks = n // CB

    def chunk(c, carry):
        m, cnt = carry
        p2c = p2t_ref[0, :, pl.ds(c * CB, CB)]
        d2 = jnp.zeros((QB, CB), jnp.float32)
        for d in range(3):
            diff = p1[:, d:d + 1] - p2c[d:d + 1, :]
            d2 = d2 + diff * diff
        mc = (d2 < rsq).astype(jnp.float32)
        cs = jnp.dot(mc, tri, preferred_element_type=jnp.float32)
        rank = cnt + cs - mc
        inc = jnp.logical_and(mc > 0.0, rank < float(NS))
        pen = jnp.where(inc, 0.0, jnp.float32(NEG))
        gc = g_ref[0, pl.ds(c * CB, CB), :]
        vals = gc[None, :, :] + pen[:, :, None]
        m = jnp.maximum(m, jnp.max(vals, axis=1))
        cnt = cnt + cs[:, CB - 1:CB]
        return m, cnt

    m0 = jnp.full((QB, C), NEG, jnp.float32)
    cnt0 = jnp.zeros((QB, 1), jnp.float32)
    m, cnt = jax.lax.fori_loop(0, nchunks, chunk, (m0, cnt0))

    g0 = g_ref[0, 0:1, :]
    m = jnp.where(cnt > 0.0, m, g0)
    out_ref[0] = jnp.maximum(m + basen_ref[0], 0.0)


@jax.jit
def kernel(P1, P2, X1, S2, W, b):
    B, N, _ = P1.shape
    f32 = jnp.float32
    zpad = jnp.zeros((B, N, 5), f32)
    P1p = jnp.concatenate([P1, zpad], axis=-1)
    P2p = jnp.concatenate([P2, zpad], axis=-1)
    P2pT = jnp.transpose(P2p, (0, 2, 1))
    S2T = jnp.transpose(S2, (0, 2, 1))
    X1T = jnp.transpose(X1, (0, 2, 1))
    WsT = jnp.transpose(W[:, :C])
    WxT = jnp.transpose(W[:, C:2 * C])
    WdT = jnp.concatenate([jnp.transpose(W[:, 2 * C:]),
                           jnp.zeros((5, C), f32)], axis=0)
    brow = b[None, :]

    g, base = pl.pallas_call(
        _pre_body,
        grid=(B,),
        in_specs=[
            pl.BlockSpec((1, N, C), lambda bi: (bi, 0, 0)),
            pl.BlockSpec((1, N, C), lambda bi: (bi, 0, 0)),
            pl.BlockSpec((1, N, 8), lambda bi: (bi, 0, 0)),
            pl.BlockSpec((1, N, 8), lambda bi: (bi, 0, 0)),
            pl.BlockSpec((C, C), lambda bi: (0, 0)),
            pl.BlockSpec((C, C), lambda bi: (0, 0)),
            pl.BlockSpec((8, C), lambda bi: (0, 0)),
            pl.BlockSpec((1, C), lambda bi: (0, 0)),
        ],
        out_specs=[
            pl.BlockSpec((1, N, C), lambda bi: (bi, 0, 0)),
            pl.BlockSpec((1, N, C), lambda bi: (bi, 0, 0)),
        ],
        out_shape=[
            jax.ShapeDtypeStruct((B, N, C), f32),
            jax.ShapeDtypeStruct((B, N, C), f32),
        ],
    )(S2T, X1T, P1p, P2p, WsT, WxT, WdT, brow)

    outn = pl.pallas_call(
        _bq_body,
        grid=(B, N // QB),
        in_specs=[
            pl.BlockSpec((1, QB, 8), lambda bi, qi: (bi, qi, 0)),
            pl.BlockSpec((1, 8, N), lambda bi, qi: (bi, 0, 0)),
            pl.BlockSpec((1, N, C), lambda bi, qi: (bi, 0, 0)),
            pl.BlockSpec((1, QB, C), lambda bi, qi: (bi, qi, 0)),
        ],
        out_specs=pl.BlockSpec((1, QB, C), lambda bi, qi: (bi, qi, 0)),
        out_shape=jax.ShapeDtypeStruct((B, N, C), f32),
    )(P1p, P2pT, g, base)

    return jnp.transpose(outn, (0, 2, 1))


# speedup vs baseline: 2.7561x; 1.0870x over previous
import functools

import jax
import jax.numpy as jnp
from jax.experimental import pallas as pl
from jax.experimental.pallas import tpu as pltpu

RADIUS = 0.2
NS = 32
C = 64
QB = 128
CB = 128
NEG = -3.0e38


def _pre_body(s2t_ref, x1t_ref, p1p_ref, p2p_ref, wst_ref, wxt_ref,
              wdt_ref, b_ref, g_ref, base_ref):
    s2t = s2t_ref[0]
    x1t = x1t_ref[0]
    p1p = p1p_ref[0]
    p2p = p2p_ref[0]
    wst = wst_ref[...]
    wxt = wxt_ref[...]
    wdt = wdt_ref[...]
    bb = b_ref[...]
    f32 = jnp.float32
    g = (jnp.dot(s2t, wst, preferred_element_type=f32)
         + jnp.dot(p2p, wdt, preferred_element_type=f32))
    base = (jnp.dot(x1t, wxt, preferred_element_type=f32)
            - jnp.dot(p1p, wdt, preferred_element_type=f32) + bb)
    g_ref[0] = g
    base_ref[0] = base


def _bq_body(p1p_ref, p2t_ref, g_ref, basen_ref, out_ref):
    p1 = p1p_ref[0]
    n = p2t_ref.shape[2]
    rsq = jnp.float32(RADIUS * RADIUS)

    ri = jax.lax.broadcasted_iota(jnp.int32, (CB, CB), 0)
    ci = jax.lax.broadcasted_iota(jnp.int32, (CB, CB), 1)
    tri = (ri <= ci).astype(jnp.float32)

    nchunks = n // CB

    def cond(carry):
        c, _, cnt = carry
        return jnp.logical_and(c < nchunks, jnp.min(cnt) < float(NS))

    def chunk(carry):
        c, m, cnt = carry
        p2c = p2t_ref[0, :, pl.ds(c * CB, CB)]
        d2 = jnp.zeros((QB, CB), jnp.float32)
        for d in range(3):
            diff = p1[:, d:d + 1] - p2c[d:d + 1, :]
            d2 = d2 + diff * diff
        mc = (d2 < rsq).astype(jnp.float32)
        cs = jnp.dot(mc, tri, preferred_element_type=jnp.float32)
        rank = cnt + cs - mc
        inc = jnp.logical_and(mc > 0.0, rank < float(NS))
        pen = jnp.where(inc, 0.0, jnp.float32(NEG))
        gc = g_ref[0, pl.ds(c * CB, CB), :]
        vals = gc[None, :, :] + pen[:, :, None]
        m = jnp.maximum(m, jnp.max(vals, axis=1))
        cnt = cnt + cs[:, CB - 1:CB]
        return c + 1, m, cnt

    m0 = jnp.full((QB, C), jnp.float32(NEG))
    cnt0 = jnp.zeros((QB, 1), jnp.float32)
    _, m, cnt = jax.lax.while_loop(cond, chunk, (0, m0, cnt0))

    g0 = g_ref[0, 0:1, :]
    m = jnp.where(cnt > 0.0, m, g0)
    out_ref[0] = jnp.maximum(m + basen_ref[0], 0.0)


@jax.jit
def kernel(P1, P2, X1, S2, W, b):
    B, N, _ = P1.shape
    f32 = jnp.float32
    zpad = jnp.zeros((B, N, 5), f32)
    P1p = jnp.concatenate([P1, zpad], axis=-1)
    P2p = jnp.concatenate([P2, zpad], axis=-1)
    P2pT = jnp.transpose(P2p, (0, 2, 1))
    S2T = jnp.transpose(S2, (0, 2, 1))
    X1T = jnp.transpose(X1, (0, 2, 1))
    WsT = jnp.transpose(W[:, :C])
    WxT = jnp.transpose(W[:, C:2 * C])
    WdT = jnp.concatenate([jnp.transpose(W[:, 2 * C:]),
                           jnp.zeros((5, C), f32)], axis=0)
    brow = b[None, :]

    g, base = pl.pallas_call(
        _pre_body,
        grid=(B,),
        in_specs=[
            pl.BlockSpec((1, N, C), lambda bi: (bi, 0, 0)),
            pl.BlockSpec((1, N, C), lambda bi: (bi, 0, 0)),
            pl.BlockSpec((1, N, 8), lambda bi: (bi, 0, 0)),
            pl.BlockSpec((1, N, 8), lambda bi: (bi, 0, 0)),
            pl.BlockSpec((C, C), lambda bi: (0, 0)),
            pl.BlockSpec((C, C), lambda bi: (0, 0)),
            pl.BlockSpec((8, C), lambda bi: (0, 0)),
            pl.BlockSpec((1, C), lambda bi: (0, 0)),
        ],
        out_specs=[
            pl.BlockSpec((1, N, C), lambda bi: (bi, 0, 0)),
            pl.BlockSpec((1, N, C), lambda bi: (bi, 0, 0)),
        ],
        out_shape=[
            jax.ShapeDtypeStruct((B, N, C), f32),
            jax.ShapeDtypeStruct((B, N, C), f32),
        ],
    )(S2T, X1T, P1p, P2p, WsT, WxT, WdT, brow)

    outn = pl.pallas_call(
        _bq_body,
        grid=(B, N // QB),
        in_specs=[
            pl.BlockSpec((1, QB, 8), lambda bi, qi: (bi, qi, 0)),
            pl.BlockSpec((1, 8, N), lambda bi, qi: (bi, 0, 0)),
            pl.BlockSpec((1, N, C), lambda bi, qi: (bi, 0, 0)),
            pl.BlockSpec((1, QB, C), lambda bi, qi: (bi, qi, 0)),
        ],
        out_specs=pl.BlockSpec((1, QB, C), lambda bi, qi: (bi, qi, 0)),
        out_shape=jax.ShapeDtypeStruct((B, N, C), f32),
    )(P1p, P2pT, g, base)

    return jnp.transpose(outn, (0, 2, 1))


# speedup vs baseline: 4.6423x; 1.6844x over previous
import functools

import jax
import jax.numpy as jnp
from jax.experimental import pallas as pl
from jax.experimental.pallas import tpu as pltpu

RADIUS = 0.2
NS = 32
C = 64
QB = 128
CB = 128
NEG = -3.0e38


def _pre_body(s2t_ref, x1t_ref, p1p_ref, p2p_ref, wst_ref, wxt_ref,
              wdt_ref, b_ref, g_ref, base_ref):
    s2t = s2t_ref[0]
    x1t = x1t_ref[0]
    p1p = p1p_ref[0]
    p2p = p2p_ref[0]
    wst = wst_ref[...]
    wxt = wxt_ref[...]
    wdt = wdt_ref[...]
    bb = b_ref[...]
    f32 = jnp.float32
    g = (jnp.dot(s2t, wst, preferred_element_type=f32)
         + jnp.dot(p2p, wdt, preferred_element_type=f32))
    base = (jnp.dot(x1t, wxt, preferred_element_type=f32)
            - jnp.dot(p1p, wdt, preferred_element_type=f32) + bb)
    g_ref[0] = g
    base_ref[0] = base


def _bq_body(p1p_ref, p2t_ref, g_ref, basen_ref, out_ref):
    p1 = p1p_ref[0]
    n = p2t_ref.shape[2]
    rsq = jnp.float32(RADIUS * RADIUS)

    ri = jax.lax.broadcasted_iota(jnp.int32, (CB, CB), 0)
    ci = jax.lax.broadcasted_iota(jnp.int32, (CB, CB), 1)
    tri = (ri <= ci).astype(jnp.float32)

    nchunks = n // CB

    def cond(carry):
        c, _, cnt = carry
        return jnp.logical_and(c < nchunks, jnp.min(cnt) < float(NS))

    def chunk(carry):
        c, m, cnt = carry
        p2c = p2t_ref[0, :, pl.ds(c * CB, CB)]
        d2 = jnp.zeros((QB, CB), jnp.float32)
        for d in range(3):
            diff = p1[:, d:d + 1] - p2c[d:d + 1, :]
            d2 = d2 + diff * diff
        mc = (d2 < rsq).astype(jnp.float32)
        cs = jnp.dot(mc, tri, preferred_element_type=jnp.float32)
        rank = cnt + cs - mc
        inc = jnp.logical_and(mc > 0.0, rank < float(NS))
        pen = jnp.where(inc, 0.0, jnp.float32(NEG)).astype(jnp.bfloat16)
        gc = g_ref[0, pl.ds(c * CB, CB), :].astype(jnp.bfloat16)
        vals = gc[None, :, :] + pen[:, :, None]
        m = jnp.maximum(m, jnp.max(vals, axis=1).astype(jnp.float32))
        cnt = cnt + cs[:, CB - 1:CB]
        return c + 1, m, cnt

    m0 = jnp.full((QB, C), jnp.float32(NEG))
    cnt0 = jnp.zeros((QB, 1), jnp.float32)
    _, m, cnt = jax.lax.while_loop(cond, chunk, (0, m0, cnt0))

    g0 = g_ref[0, 0:1, :]
    m = jnp.where(cnt > 0.0, m, g0)
    out_ref[0] = jnp.maximum(m + basen_ref[0], 0.0)


@jax.jit
def kernel(P1, P2, X1, S2, W, b):
    B, N, _ = P1.shape
    f32 = jnp.float32
    zpad = jnp.zeros((B, N, 5), f32)
    P1p = jnp.concatenate([P1, zpad], axis=-1)
    P2p = jnp.concatenate([P2, zpad], axis=-1)
    P2pT = jnp.transpose(P2p, (0, 2, 1))
    S2T = jnp.transpose(S2, (0, 2, 1))
    X1T = jnp.transpose(X1, (0, 2, 1))
    WsT = jnp.transpose(W[:, :C])
    WxT = jnp.transpose(W[:, C:2 * C])
    WdT = jnp.concatenate([jnp.transpose(W[:, 2 * C:]),
                           jnp.zeros((5, C), f32)], axis=0)
    brow = b[None, :]

    g, base = pl.pallas_call(
        _pre_body,
        grid=(B,),
        in_specs=[
            pl.BlockSpec((1, N, C), lambda bi: (bi, 0, 0)),
            pl.BlockSpec((1, N, C), lambda bi: (bi, 0, 0)),
            pl.BlockSpec((1, N, 8), lambda bi: (bi, 0, 0)),
            pl.BlockSpec((1, N, 8), lambda bi: (bi, 0, 0)),
            pl.BlockSpec((C, C), lambda bi: (0, 0)),
            pl.BlockSpec((C, C), lambda bi: (0, 0)),
            pl.BlockSpec((8, C), lambda bi: (0, 0)),
            pl.BlockSpec((1, C), lambda bi: (0, 0)),
        ],
        out_specs=[
            pl.BlockSpec((1, N, C), lambda bi: (bi, 0, 0)),
            pl.BlockSpec((1, N, C), lambda bi: (bi, 0, 0)),
        ],
        out_shape=[
            jax.ShapeDtypeStruct((B, N, C), f32),
            jax.ShapeDtypeStruct((B, N, C), f32),
        ],
    )(S2T, X1T, P1p, P2p, WsT, WxT, WdT, brow)

    outn = pl.pallas_call(
        _bq_body,
        grid=(B, N // QB),
        in_specs=[
            pl.BlockSpec((1, QB, 8), lambda bi, qi: (bi, qi, 0)),
            pl.BlockSpec((1, 8, N), lambda bi, qi: (bi, 0, 0)),
            pl.BlockSpec((1, N, C), lambda bi, qi: (bi, 0, 0)),
            pl.BlockSpec((1, QB, C), lambda bi, qi: (bi, qi, 0)),
        ],
        out_specs=pl.BlockSpec((1, QB, C), lambda bi, qi: (bi, qi, 0)),
        out_shape=jax.ShapeDtypeStruct((B, N, C), f32),
    )(P1p, P2pT, g, base)

    return jnp.transpose(outn, (0, 2, 1))


# speedup vs baseline: 7.5501x; 1.6264x over previous
import functools

import jax
import jax.numpy as jnp
from jax.experimental import pallas as pl
from jax.experimental.pallas import tpu as pltpu

RADIUS = 0.2
NS = 32
C = 64
QB = 128
CB = 128
NEG = -3.0e38


def _pre_body(s2t_ref, x1t_ref, p1p_ref, p2p_ref, wst_ref, wxt_ref,
              wdt_ref, b_ref, g_ref, base_ref):
    s2t = s2t_ref[0]
    x1t = x1t_ref[0]
    p1p = p1p_ref[0]
    p2p = p2p_ref[0]
    wst = wst_ref[...]
    wxt = wxt_ref[...]
    wdt = wdt_ref[...]
    bb = b_ref[...]
    f32 = jnp.float32
    g = (jnp.dot(s2t, wst, preferred_element_type=f32)
         + jnp.dot(p2p, wdt, preferred_element_type=f32))
    base = (jnp.dot(x1t, wxt, preferred_element_type=f32)
            - jnp.dot(p1p, wdt, preferred_element_type=f32) + bb)
    g_ref[0] = g
    base_ref[0] = base


def _bq_body(p1p_ref, p2t_ref, g_ref, basen_ref, out_ref):
    p1 = p1p_ref[0]
    n = p2t_ref.shape[2]
    rsq = jnp.float32(RADIUS * RADIUS)

    ri = jax.lax.broadcasted_iota(jnp.int32, (CB, CB), 0)
    ci = jax.lax.broadcasted_iota(jnp.int32, (CB, CB), 1)
    tri = (ri <= ci).astype(jnp.float32)

    nchunks = n // CB

    def cond(carry):
        c, _, cnt = carry
        return jnp.logical_and(c < nchunks, jnp.min(cnt) < float(NS))

    def chunk(carry):
        c, m, cnt = carry
        p2c = p2t_ref[0, :, pl.ds(c * CB, CB)]
        d2 = jnp.zeros((QB, CB), jnp.float32)
        for d in range(3):
            diff = p1[:, d:d + 1] - p2c[d:d + 1, :]
            d2 = d2 + diff * diff
        mc = (d2 < rsq).astype(jnp.float32)
        cs = jnp.dot(mc, tri, preferred_element_type=jnp.float32)
        rank = cnt + cs - mc
        inc = jnp.logical_and(mc > 0.0, rank < float(NS))
        pen = jnp.where(inc, 0.0, jnp.float32(NEG)).astype(jnp.bfloat16)
        gc = g_ref[0, pl.ds(c * CB, CB), :].astype(jnp.bfloat16)
        vals = gc[None, :, :] + pen[:, :, None]
        m = jnp.maximum(m, jnp.max(vals, axis=1).astype(jnp.float32))
        cnt = cnt + cs[:, CB - 1:CB]
        return c + 1, m, cnt

    m0 = jnp.full((QB, C), jnp.float32(NEG))
    cnt0 = jnp.zeros((QB, 1), jnp.float32)
    _, m, cnt = jax.lax.while_loop(cond, chunk, (0, m0, cnt0))

    g0 = g_ref[0, 0:1, :]
    m = jnp.where(cnt > 0.0, m, g0)
    out_ref[0] = jnp.maximum(m + basen_ref[0], 0.0)


@jax.jit
def kernel(P1, P2, X1, S2, W, b):
    B, N, _ = P1.shape
    f32 = jnp.float32

    ext = (jnp.minimum(P1 + RADIUS, 1.0)
           - jnp.maximum(P1 - RADIUS, 0.0))
    proxy = ext[..., 0] * ext[..., 1] * ext[..., 2]
    order = jnp.argsort(-proxy, axis=1)
    P1 = jnp.take_along_axis(P1, order[:, :, None], axis=1)

    zpad = jnp.zeros((B, N, 5), f32)
    P1p = jnp.concatenate([P1, zpad], axis=-1)
    P2p = jnp.concatenate([P2, zpad], axis=-1)
    P2pT = jnp.transpose(P2p, (0, 2, 1))
    S2T = jnp.transpose(S2, (0, 2, 1))
    X1T = jnp.take_along_axis(jnp.transpose(X1, (0, 2, 1)),
                              order[:, :, None], axis=1)
    WsT = jnp.transpose(W[:, :C])
    WxT = jnp.transpose(W[:, C:2 * C])
    WdT = jnp.concatenate([jnp.transpose(W[:, 2 * C:]),
                           jnp.zeros((5, C), f32)], axis=0)
    brow = b[None, :]

    g, base = pl.pallas_call(
        _pre_body,
        grid=(B,),
        in_specs=[
            pl.BlockSpec((1, N, C), lambda bi: (bi, 0, 0)),
            pl.BlockSpec((1, N, C), lambda bi: (bi, 0, 0)),
            pl.BlockSpec((1, N, 8), lambda bi: (bi, 0, 0)),
            pl.BlockSpec((1, N, 8), lambda bi: (bi, 0, 0)),
            pl.BlockSpec((C, C), lambda bi: (0, 0)),
            pl.BlockSpec((C, C), lambda bi: (0, 0)),
            pl.BlockSpec((8, C), lambda bi: (0, 0)),
            pl.BlockSpec((1, C), lambda bi: (0, 0)),
        ],
        out_specs=[
            pl.BlockSpec((1, N, C), lambda bi: (bi, 0, 0)),
            pl.BlockSpec((1, N, C), lambda bi: (bi, 0, 0)),
        ],
        out_shape=[
            jax.ShapeDtypeStruct((B, N, C), f32),
            jax.ShapeDtypeStruct((B, N, C), f32),
        ],
    )(S2T, X1T, P1p, P2p, WsT, WxT, WdT, brow)

    outn = pl.pallas_call(
        _bq_body,
        grid=(B, N // QB),
        in_specs=[
            pl.BlockSpec((1, QB, 8), lambda bi, qi: (bi, qi, 0)),
            pl.BlockSpec((1, 8, N), lambda bi, qi: (bi, 0, 0)),
            pl.BlockSpec((1, N, C), lambda bi, qi: (bi, 0, 0)),
            pl.BlockSpec((1, QB, C), lambda bi, qi: (bi, qi, 0)),
        ],
        out_specs=pl.BlockSpec((1, QB, C), lambda bi, qi: (bi, qi, 0)),
        out_shape=jax.ShapeDtypeStruct((B, N, C), f32),
    )(P1p, P2pT, g, base)

    outu = jnp.zeros_like(outn).at[
        jnp.arange(B)[:, None], order].set(outn)
    return jnp.transpose(outu, (0, 2, 1))


# speedup vs baseline: 8.4789x; 1.1230x over previous
import functools

import jax
import jax.numpy as jnp
from jax.experimental import pallas as pl
from jax.experimental.pallas import tpu as pltpu

RADIUS = 0.2
NS = 32
C = 64
QB = 128
CB = 256
NEG = -3.0e38


def _pre_body(s2t_ref, x1t_ref, p1p_ref, p2p_ref, wst_ref, wxt_ref,
              wdt_ref, b_ref, g_ref, base_ref):
    s2t = s2t_ref[0]
    x1t = x1t_ref[0]
    p1p = p1p_ref[0]
    p2p = p2p_ref[0]
    wst = wst_ref[...]
    wxt = wxt_ref[...]
    wdt = wdt_ref[...]
    bb = b_ref[...]
    f32 = jnp.float32
    g = (jnp.dot(s2t, wst, preferred_element_type=f32)
         + jnp.dot(p2p, wdt, preferred_element_type=f32))
    base = (jnp.dot(x1t, wxt, preferred_element_type=f32)
            - jnp.dot(p1p, wdt, preferred_element_type=f32) + bb)
    g_ref[0] = g
    base_ref[0] = base


def _bq_body(p1p_ref, p2t_ref, g_ref, basen_ref, out_ref):
    p1 = p1p_ref[0]
    n = p2t_ref.shape[2]
    rsq = jnp.float32(RADIUS * RADIUS)

    ri = jax.lax.broadcasted_iota(jnp.int32, (CB, CB), 0)
    ci = jax.lax.broadcasted_iota(jnp.int32, (CB, CB), 1)
    tri = (ri <= ci).astype(jnp.float32)

    nchunks = n // CB

    def cond(carry):
        c, _, cnt = carry
        return jnp.logical_and(c < nchunks, jnp.min(cnt) < float(NS))

    def chunk(carry):
        c, m, cnt = carry
        p2c = p2t_ref[0, :, pl.ds(c * CB, CB)]
        d2 = jnp.zeros((QB, CB), jnp.float32)
        for d in range(3):
            diff = p1[:, d:d + 1] - p2c[d:d + 1, :]
            d2 = d2 + diff * diff
        mc = (d2 < rsq).astype(jnp.float32)
        cs = jnp.dot(mc, tri, preferred_element_type=jnp.float32)
        rank = cnt + cs - mc
        inc = jnp.logical_and(mc > 0.0, rank < float(NS))
        pen = jnp.where(inc, 0.0, jnp.float32(NEG)).astype(jnp.bfloat16)
        gc = g_ref[0, pl.ds(c * CB, CB), :].astype(jnp.bfloat16)
        vals = gc[None, :, :] + pen[:, :, None]
        m = jnp.maximum(m, jnp.max(vals, axis=1).astype(jnp.float32))
        cnt = cnt + cs[:, CB - 1:CB]
        return c + 1, m, cnt

    m0 = jnp.full((QB, C), jnp.float32(NEG))
    cnt0 = jnp.zeros((QB, 1), jnp.float32)
    _, m, cnt = jax.lax.while_loop(cond, chunk, (0, m0, cnt0))

    g0 = g_ref[0, 0:1, :]
    m = jnp.where(cnt > 0.0, m, g0)
    out_ref[0] = jnp.maximum(m + basen_ref[0], 0.0)


@jax.jit
def kernel(P1, P2, X1, S2, W, b):
    B, N, _ = P1.shape
    f32 = jnp.float32

    ext = (jnp.minimum(P1 + RADIUS, 1.0)
           - jnp.maximum(P1 - RADIUS, 0.0))
    proxy = ext[..., 0] * ext[..., 1] * ext[..., 2]
    order = jnp.argsort(-proxy, axis=1)
    P1 = jnp.take_along_axis(P1, order[:, :, None], axis=1)

    zpad = jnp.zeros((B, N, 5), f32)
    P1p = jnp.concatenate([P1, zpad], axis=-1)
    P2p = jnp.concatenate([P2, zpad], axis=-1)
    P2pT = jnp.transpose(P2p, (0, 2, 1))
    S2T = jnp.transpose(S2, (0, 2, 1))
    X1T = jnp.take_along_axis(jnp.transpose(X1, (0, 2, 1)),
                              order[:, :, None], axis=1)
    WsT = jnp.transpose(W[:, :C])
    WxT = jnp.transpose(W[:, C:2 * C])
    WdT = jnp.concatenate([jnp.transpose(W[:, 2 * C:]),
                           jnp.zeros((5, C), f32)], axis=0)
    brow = b[None, :]

    g, base = pl.pallas_call(
        _pre_body,
        grid=(B,),
        in_specs=[
            pl.BlockSpec((1, N, C), lambda bi: (bi, 0, 0)),
            pl.BlockSpec((1, N, C), lambda bi: (bi, 0, 0)),
            pl.BlockSpec((1, N, 8), lambda bi: (bi, 0, 0)),
            pl.BlockSpec((1, N, 8), lambda bi: (bi, 0, 0)),
            pl.BlockSpec((C, C), lambda bi: (0, 0)),
            pl.BlockSpec((C, C), lambda bi: (0, 0)),
            pl.BlockSpec((8, C), lambda bi: (0, 0)),
            pl.BlockSpec((1, C), lambda bi: (0, 0)),
        ],
        out_specs=[
            pl.BlockSpec((1, N, C), lambda bi: (bi, 0, 0)),
            pl.BlockSpec((1, N, C), lambda bi: (bi, 0, 0)),
        ],
        out_shape=[
            jax.ShapeDtypeStruct((B, N, C), f32),
            jax.ShapeDtypeStruct((B, N, C), f32),
        ],
    )(S2T, X1T, P1p, P2p, WsT, WxT, WdT, brow)

    outn = pl.pallas_call(
        _bq_body,
        grid=(B, N // QB),
        in_specs=[
            pl.BlockSpec((1, QB, 8), lambda bi, qi: (bi, qi, 0)),
            pl.BlockSpec((1, 8, N), lambda bi, qi: (bi, 0, 0)),
            pl.BlockSpec((1, N, C), lambda bi, qi: (bi, 0, 0)),
            pl.BlockSpec((1, QB, C), lambda bi, qi: (bi, qi, 0)),
        ],
        out_specs=pl.BlockSpec((1, QB, C), lambda bi, qi: (bi, qi, 0)),
        out_shape=jax.ShapeDtypeStruct((B, N, C), f32),
    )(P1p, P2pT, g, base)

    outu = jnp.zeros_like(outn).at[
        jnp.arange(B)[:, None], order].set(outn)
    return jnp.transpose(outu, (0, 2, 1))


# speedup vs baseline: 8.7320x; 1.0299x over previous
import functools

import jax
import jax.numpy as jnp
from jax.experimental import pallas as pl
from jax.experimental.pallas import tpu as pltpu

RADIUS = 0.2
NS = 32
C = 64
QB = 128
CB = 512
NEG = -3.0e38


def _pre_body(s2t_ref, x1t_ref, p1p_ref, p2p_ref, wst_ref, wxt_ref,
              wdt_ref, b_ref, g_ref, base_ref):
    s2t = s2t_ref[0]
    x1t = x1t_ref[0]
    p1p = p1p_ref[0]
    p2p = p2p_ref[0]
    wst = wst_ref[...]
    wxt = wxt_ref[...]
    wdt = wdt_ref[...]
    bb = b_ref[...]
    f32 = jnp.float32
    g = (jnp.dot(s2t, wst, preferred_element_type=f32)
         + jnp.dot(p2p, wdt, preferred_element_type=f32))
    base = (jnp.dot(x1t, wxt, preferred_element_type=f32)
            - jnp.dot(p1p, wdt, preferred_element_type=f32) + bb)
    g_ref[0] = g
    base_ref[0] = base


def _bq_body(p1p_ref, p2t_ref, g_ref, basen_ref, out_ref):
    p1 = p1p_ref[0]
    n = p2t_ref.shape[2]
    rsq = jnp.float32(RADIUS * RADIUS)

    ri = jax.lax.broadcasted_iota(jnp.int32, (CB, CB), 0)
    ci = jax.lax.broadcasted_iota(jnp.int32, (CB, CB), 1)
    tri = (ri <= ci).astype(jnp.float32)

    nchunks = n // CB

    def cond(carry):
        c, _, cnt = carry
        return jnp.logical_and(c < nchunks, jnp.min(cnt) < float(NS))

    def chunk(carry):
        c, m, cnt = carry
        p2c = p2t_ref[0, :, pl.ds(c * CB, CB)]
        d2 = jnp.zeros((QB, CB), jnp.float32)
        for d in range(3):
            diff = p1[:, d:d + 1] - p2c[d:d + 1, :]
            d2 = d2 + diff * diff
        mc = (d2 < rsq).astype(jnp.float32)
        cs = jnp.dot(mc, tri, preferred_element_type=jnp.float32)
        rank = cnt + cs - mc
        inc = jnp.logical_and(mc > 0.0, rank < float(NS))
        pen = jnp.where(inc, 0.0, jnp.float32(NEG)).astype(jnp.bfloat16)
        gc = g_ref[0, pl.ds(c * CB, CB), :].astype(jnp.bfloat16)
        vals = gc[None, :, :] + pen[:, :, None]
        m = jnp.maximum(m, jnp.max(vals, axis=1).astype(jnp.float32))
        cnt = cnt + cs[:, CB - 1:CB]
        return c + 1, m, cnt

    m0 = jnp.full((QB, C), jnp.float32(NEG))
    cnt0 = jnp.zeros((QB, 1), jnp.float32)
    _, m, cnt = jax.lax.while_loop(cond, chunk, (0, m0, cnt0))

    g0 = g_ref[0, 0:1, :]
    m = jnp.where(cnt > 0.0, m, g0)
    out_ref[0] = jnp.maximum(m + basen_ref[0], 0.0)


@jax.jit
def kernel(P1, P2, X1, S2, W, b):
    B, N, _ = P1.shape
    f32 = jnp.float32

    ext = (jnp.minimum(P1 + RADIUS, 1.0)
           - jnp.maximum(P1 - RADIUS, 0.0))
    proxy = ext[..., 0] * ext[..., 1] * ext[..., 2]
    order = jnp.argsort(-proxy, axis=1)
    P1 = jnp.take_along_axis(P1, order[:, :, None], axis=1)

    zpad = jnp.zeros((B, N, 5), f32)
    P1p = jnp.concatenate([P1, zpad], axis=-1)
    P2p = jnp.concatenate([P2, zpad], axis=-1)
    P2pT = jnp.transpose(P2p, (0, 2, 1))
    S2T = jnp.transpose(S2, (0, 2, 1))
    X1T = jnp.take_along_axis(jnp.transpose(X1, (0, 2, 1)),
                              order[:, :, None], axis=1)
    WsT = jnp.transpose(W[:, :C])
    WxT = jnp.transpose(W[:, C:2 * C])
    WdT = jnp.concatenate([jnp.transpose(W[:, 2 * C:]),
                           jnp.zeros((5, C), f32)], axis=0)
    brow = b[None, :]

    g, base = pl.pallas_call(
        _pre_body,
        grid=(B,),
        in_specs=[
            pl.BlockSpec((1, N, C), lambda bi: (bi, 0, 0)),
            pl.BlockSpec((1, N, C), lambda bi: (bi, 0, 0)),
            pl.BlockSpec((1, N, 8), lambda bi: (bi, 0, 0)),
            pl.BlockSpec((1, N, 8), lambda bi: (bi, 0, 0)),
            pl.BlockSpec((C, C), lambda bi: (0, 0)),
            pl.BlockSpec((C, C), lambda bi: (0, 0)),
            pl.BlockSpec((8, C), lambda bi: (0, 0)),
            pl.BlockSpec((1, C), lambda bi: (0, 0)),
        ],
        out_specs=[
            pl.BlockSpec((1, N, C), lambda bi: (bi, 0, 0)),
            pl.BlockSpec((1, N, C), lambda bi: (bi, 0, 0)),
        ],
        out_shape=[
            jax.ShapeDtypeStruct((B, N, C), f32),
            jax.ShapeDtypeStruct((B, N, C), f32),
        ],
    )(S2T, X1T, P1p, P2p, WsT, WxT, WdT, brow)

    outn = pl.pallas_call(
        _bq_body,
        grid=(B, N // QB),
        in_specs=[
            pl.BlockSpec((1, QB, 8), lambda bi, qi: (bi, qi, 0)),
            pl.BlockSpec((1, 8, N), lambda bi, qi: (bi, 0, 0)),
            pl.BlockSpec((1, N, C), lambda bi, qi: (bi, 0, 0)),
            pl.BlockSpec((1, QB, C), lambda bi, qi: (bi, qi, 0)),
        ],
        out_specs=pl.BlockSpec((1, QB, C), lambda bi, qi: (bi, qi, 0)),
        out_shape=jax.ShapeDtypeStruct((B, N, C), f32),
    )(P1p, P2pT, g, base)

    outu = jnp.zeros_like(outn).at[
        jnp.arange(B)[:, None], order].set(outn)
    return jnp.transpose(outu, (0, 2, 1))
